# Initial kernel scaffold; baseline (speedup 1.0000x reference)
import jax
import jax.numpy as jnp
from jax.experimental import pallas as pl


def kernel(xyz, bn_w0, bn_b0, bn_w1, bn_b1, bn_w2, bn_b2, bn_w3, bn_b3):
    raise NotImplementedError("write your pallas kernel here")


# speedup vs baseline: 1.2004x; 1.2004x over previous
import math
from functools import partial

import jax
import jax.numpy as jnp
import numpy as np
from jax.experimental import pallas as pl
from jax.experimental.pallas import tpu as pltpu

B = 8
N0 = 2048
INIT_DIM = 64
K = 24
ALPHA = 1000.0
BETA = 100.0
STAGE_DIMS = [128, 256, 512, 1024]


def _spe_maps(out_dim):
    in_dim = 3
    feat_dim = math.ceil(out_dim / (in_dim * 2))
    feat_num = feat_dim * 2 * in_dim
    out_idx = np.linspace(0, feat_num - 1, out_dim).astype(np.int32)
    dim_embed = np.power(np.float32(ALPHA),
                         np.arange(feat_dim, dtype=np.float32) / feat_dim)
    coord = out_idx // (2 * feat_dim)
    rem = out_idx % (2 * feat_dim)
    freq = rem // 2
    issin = (rem % 2 == 0)
    m0 = (coord == 0).astype(np.float32)
    m1 = (coord == 1).astype(np.float32)
    m2 = (coord == 2).astype(np.float32)
    de = dim_embed[freq].astype(np.float32)
    return (m0.reshape(1, out_dim), m1.reshape(1, out_dim),
            m2.reshape(1, out_dim), de.reshape(1, out_dim),
            issin.astype(np.float32).reshape(1, out_dim))


def _fps_body(xt_ref, cent_ref, *, npoint, n):
    X = xt_ref[0]
    Y = xt_ref[1]
    Z = xt_ref[2]
    lane = jax.lax.broadcasted_iota(jnp.int32, (B, n), 1)
    col = jax.lax.broadcasted_iota(jnp.int32, (B, npoint), 1)

    def body(i, carry):
        dist, far, cent = carry
        cent = cent + (col == i).astype(jnp.int32) * far
        mask = lane == far
        cx = jnp.sum(jnp.where(mask, X, 0.0), axis=1, keepdims=True)
        cy = jnp.sum(jnp.where(mask, Y, 0.0), axis=1, keepdims=True)
        cz = jnp.sum(jnp.where(mask, Z, 0.0), axis=1, keepdims=True)
        dx = X - cx
        dy = Y - cy
        dz = Z - cz
        d = dx * dx + dy * dy + dz * dz
        dist = jnp.minimum(dist, d)
        m = jnp.max(dist, axis=1, keepdims=True)
        far2 = jnp.min(jnp.where(dist == m, lane, n), axis=1, keepdims=True)
        return dist, far2.astype(jnp.int32), cent

    cent_ref[:, :] = jnp.zeros((B, npoint), jnp.int32)
    cent0 = cent_ref[:, :]
    dist0 = jnp.minimum(X * X * 0.0 + 1e10, 1e10)
    far0 = jnp.zeros((B, 1), jnp.int32)
    _, _, cent = jax.lax.fori_loop(0, npoint, body, (dist0, far0, cent0))
    cent_ref[:, :] = cent


def _fps(xyz, npoint):
    n = xyz.shape[1]
    xt = jnp.transpose(xyz, (2, 0, 1))
    return pl.pallas_call(
        partial(_fps_body, npoint=npoint, n=n),
        out_shape=jax.ShapeDtypeStruct((B, npoint), jnp.int32),
    )(xt)


_TSK = 8


def _knn_body(xt_ref, q_ref, out_ref, *, n):
    xt = xt_ref[0]
    xr = xt[0:1, :]
    yr = xt[1:2, :]
    zr = xt[2:3, :]
    q = q_ref[0]
    qx = q[:, 0:1]
    qy = q[:, 1:2]
    qz = q[:, 2:3]
    xsq = xr * xr + yr * yr + zr * zr
    qsq = qx * qx + qy * qy + qz * qz
    dot = jax.lax.dot_general(q, xt, (((1,), (0,)), ((), ())),
                              preferred_element_type=jnp.float32)
    d = (qsq - 2.0 * dot) + xsq
    lane = jax.lax.broadcasted_iota(jnp.int32, (_TSK, n), 1)
    kcol = jax.lax.broadcasted_iota(jnp.int32, (_TSK, K), 1)
    inf = jnp.float32(np.inf)

    def body(k, carry):
        d, acc = carry
        m = jnp.min(d, axis=1, keepdims=True)
        idx = jnp.min(jnp.where(d == m, lane, n), axis=1, keepdims=True)
        acc = acc + (kcol == k).astype(jnp.int32) * idx.astype(jnp.int32)
        return jnp.where(lane == idx, inf, d), acc

    out_ref[0] = jnp.zeros((_TSK, K), jnp.int32)
    acc0 = out_ref[0]
    _, acc = jax.lax.fori_loop(0, K, body, (d, acc0))
    out_ref[0] = acc


def _knn(xyz, new_xyz):
    n = xyz.shape[1]
    s = new_xyz.shape[1]
    xt = jnp.transpose(xyz, (0, 2, 1))
    return pl.pallas_call(
        partial(_knn_body, n=n),
        grid=(B, s // _TSK),
        in_specs=[
            pl.BlockSpec((1, 3, n), lambda b, i: (b, 0, 0)),
            pl.BlockSpec((1, _TSK, 3), lambda b, i: (b, i, 0)),
        ],
        out_specs=pl.BlockSpec((1, _TSK, K), lambda b, i: (b, i, 0)),
        out_shape=jax.ShapeDtypeStruct((B, s, K), jnp.int32),
    )(xt, new_xyz)


_TSF = 8


def _fuse_body(fk_ref, xk_ref, fs_ref, xs_ref, den_ref, m0_ref, m1_ref,
               m2_ref, de_ref, issin_ref, out_ref, *, c):
    c2 = 2 * c
    fk = fk_ref[0]
    xk = xk_ref[0]
    fs = fs_ref[0]
    xs = xs_ref[0]
    xden = den_ref[0:1, 0:1]
    fden = den_ref[0:1, 1:2]
    xkn = (xk - xs[:, None, :]) / xden[None]
    fkn = (fk - fs[:, None, :]) / fden[None]
    m0 = m0_ref[:, :].reshape(1, 1, c2)
    m1 = m1_ref[:, :].reshape(1, 1, c2)
    m2 = m2_ref[:, :].reshape(1, 1, c2)
    de = de_ref[:, :].reshape(1, 1, c2)
    issin = issin_ref[:, :].reshape(1, 1, c2) != 0.0
    xsel = xkn[:, :, 0:1] * m0 + xkn[:, :, 1:2] * m1 + xkn[:, :, 2:3] * m2
    div = (BETA * xsel) / de
    pe = jnp.where(issin, jnp.sin(div), jnp.cos(div))
    pe1 = pe[:, :, :c]
    pe2 = pe[:, :, c:]
    w1 = (fkn + pe1) * pe1
    w2 = (fs[:, None, :] + pe2) * pe2
    agg1 = jnp.max(w1, axis=1) + jnp.mean(w1, axis=1)
    agg2 = jnp.max(w2, axis=1) + jnp.mean(w2, axis=1)
    out_ref[0] = jnp.concatenate([agg1, agg2], axis=-1)


def _fuse(fk, xk, fs, xs, dens, maps, c):
    s = fs.shape[1]
    c2 = 2 * c
    m0, m1, m2, de, issin = (jnp.asarray(a) for a in maps)
    return pl.pallas_call(
        partial(_fuse_body, c=c),
        grid=(B, s // _TSF),
        in_specs=[
            pl.BlockSpec((1, _TSF, K, c), lambda b, i: (b, i, 0, 0)),
            pl.BlockSpec((1, _TSF, K, 3), lambda b, i: (b, i, 0, 0)),
            pl.BlockSpec((1, _TSF, c), lambda b, i: (b, i, 0)),
            pl.BlockSpec((1, _TSF, 3), lambda b, i: (b, i, 0)),
            pl.BlockSpec((1, 2), lambda b, i: (0, 0)),
            pl.BlockSpec((1, c2), lambda b, i: (0, 0)),
            pl.BlockSpec((1, c2), lambda b, i: (0, 0)),
            pl.BlockSpec((1, c2), lambda b, i: (0, 0)),
            pl.BlockSpec((1, c2), lambda b, i: (0, 0)),
            pl.BlockSpec((1, c2), lambda b, i: (0, 0)),
        ],
        out_specs=pl.BlockSpec((1, _TSF, c2), lambda b, i: (b, i, 0)),
        out_shape=jax.ShapeDtypeStruct((B, s, c2), jnp.float32),
    )(fk, xk, fs, xs, dens, m0, m1, m2, de, issin)


def _index_points(points, idx):
    bidx = jnp.arange(B).reshape((B,) + (1,) * (idx.ndim - 1))
    return points[bidx, idx]


def _spe_embed_jnp(xyz, out_dim):
    in_dim = 3
    feat_dim = math.ceil(out_dim / (in_dim * 2))
    feat_num = feat_dim * 2 * in_dim
    out_idx = jnp.asarray(np.linspace(0, feat_num - 1, out_dim).astype(np.int32))
    feat_val = jnp.arange(feat_dim, dtype=jnp.float32)
    dim_embed = jnp.power(jnp.float32(ALPHA), feat_val / feat_dim)
    embeds = []
    for i in range(in_dim):
        tmp = BETA * xyz[..., i:i + 1]
        div = tmp / dim_embed
        e = jnp.stack([jnp.sin(div), jnp.cos(div)], axis=-1)
        e = e.reshape(e.shape[:-2] + (feat_dim * 2,))
        embeds.append(e)
    pe = jnp.concatenate(embeds, axis=-1)
    return jnp.take(pe, out_idx, axis=-1)


def _bn_gelu(x, gamma, beta_p):
    m = jnp.mean(x, axis=(0, 1), keepdims=True)
    v = jnp.var(x, axis=(0, 1), keepdims=True)
    y = gamma * (x - m) / jnp.sqrt(v + 1e-5) + beta_p
    return jax.nn.gelu(y, approximate=False)


def _forward(xyz, bn_params):
    feat = _spe_embed_jnp(xyz, INIT_DIM)
    out_dim = INIT_DIM
    stage_points = N0
    for i in range(4):
        out_dim = out_dim * 2
        stage_points = stage_points // 2
        c = out_dim // 2
        fps_idx = _fps(xyz, stage_points)
        xyz_s = _index_points(xyz, fps_idx)
        feat_s = _index_points(feat, fps_idx)
        idx_knn = _knn(xyz, xyz_s)
        xyz_knn = _index_points(xyz, idx_knn)
        feat_knn = _index_points(feat, idx_knn)
        xyz_std = jnp.std(xyz_knn - xyz_s[:, :, None, :], ddof=1)
        feat_std = jnp.std(feat_knn - feat_s[:, :, None, :], ddof=1)
        dens = jnp.stack([xyz_std + 1e-5, feat_std + 1e-5]).reshape(1, 2)
        maps = _spe_maps(out_dim)
        agg = _fuse(feat_knn, xyz_knn, feat_s, xyz_s, dens, maps, c)
        feat = _bn_gelu(agg, bn_params[i][0], bn_params[i][1])
        xyz = xyz_s
    return jnp.max(feat, axis=-2) + jnp.mean(feat, axis=-2)


@jax.jit
def kernel(xyz, bn_w0, bn_b0, bn_w1, bn_b1, bn_w2, bn_b2, bn_w3, bn_b3):
    bn_params = [(bn_w0, bn_b0), (bn_w1, bn_b1), (bn_w2, bn_b2),
                 (bn_w3, bn_b3)]
    return _forward(xyz, bn_params)


# speedup vs baseline: 1.9776x; 1.6474x over previous
import math
from functools import partial

import jax
import jax.numpy as jnp
import numpy as np
from jax.experimental import pallas as pl
from jax.experimental.pallas import tpu as pltpu

B = 8
N0 = 2048
INIT_DIM = 64
K = 24
ALPHA = 1000.0
BETA = 100.0
STAGE_DIMS = [128, 256, 512, 1024]


def _spe_maps(out_dim):
    in_dim = 3
    feat_dim = math.ceil(out_dim / (in_dim * 2))
    feat_num = feat_dim * 2 * in_dim
    out_idx = np.linspace(0, feat_num - 1, out_dim).astype(np.int32)
    dim_embed = np.power(np.float32(ALPHA),
                         np.arange(feat_dim, dtype=np.float32) / feat_dim)
    coord = out_idx // (2 * feat_dim)
    rem = out_idx % (2 * feat_dim)
    freq = rem // 2
    issin = (rem % 2 == 0)
    m0 = (coord == 0).astype(np.float32)
    m1 = (coord == 1).astype(np.float32)
    m2 = (coord == 2).astype(np.float32)
    de = dim_embed[freq].astype(np.float32)
    return (m0.reshape(1, out_dim), m1.reshape(1, out_dim),
            m2.reshape(1, out_dim), de.reshape(1, out_dim),
            issin.astype(np.float32).reshape(1, out_dim))


def _fps_body(xt_ref, cent_ref, *, npoint, n):
    X = xt_ref[0]
    Y = xt_ref[1]
    Z = xt_ref[2]
    lane = jax.lax.broadcasted_iota(jnp.int32, (B, n), 1)
    col = jax.lax.broadcasted_iota(jnp.int32, (B, npoint), 1)

    def body(i, carry):
        dist, far, cent = carry
        cent = cent + (col == i).astype(jnp.int32) * far
        mask = lane == far
        cx = jnp.sum(jnp.where(mask, X, 0.0), axis=1, keepdims=True)
        cy = jnp.sum(jnp.where(mask, Y, 0.0), axis=1, keepdims=True)
        cz = jnp.sum(jnp.where(mask, Z, 0.0), axis=1, keepdims=True)
        dx = X - cx
        dy = Y - cy
        dz = Z - cz
        d = dx * dx + dy * dy + dz * dz
        dist = jnp.minimum(dist, d)
        m = jnp.max(dist, axis=1, keepdims=True)
        far2 = jnp.min(jnp.where(dist == m, lane, n), axis=1, keepdims=True)
        return dist, far2.astype(jnp.int32), cent

    cent_ref[:, :] = jnp.zeros((B, npoint), jnp.int32)
    cent0 = cent_ref[:, :]
    dist0 = jnp.minimum(X * X * 0.0 + 1e10, 1e10)
    far0 = jnp.zeros((B, 1), jnp.int32)
    _, _, cent = jax.lax.fori_loop(0, npoint, body, (dist0, far0, cent0))
    cent_ref[:, :] = cent


def _fps(xyz, npoint):
    n = xyz.shape[1]
    xt = jnp.transpose(xyz, (2, 0, 1))
    return pl.pallas_call(
        partial(_fps_body, npoint=npoint, n=n),
        out_shape=jax.ShapeDtypeStruct((B, npoint), jnp.int32),
    )(xt)


_TSK = 128


def _knn_body(xt_ref, q_ref, out_ref, d_scr, *, n):
    xt = xt_ref[0]
    xr = xt[0:1, :]
    yr = xt[1:2, :]
    zr = xt[2:3, :]
    q = q_ref[0]
    qx = q[:, 0:1]
    qy = q[:, 1:2]
    qz = q[:, 2:3]
    xsq = xr * xr + yr * yr + zr * zr
    qsq = qx * qx + qy * qy + qz * qz
    dot = jax.lax.dot_general(q, xt, (((1,), (0,)), ((), ())),
                              preferred_element_type=jnp.float32)
    d_scr[:, :] = (qsq - 2.0 * dot) + xsq
    lane = jax.lax.broadcasted_iota(jnp.int32, (8, n), 1)
    kcol = jax.lax.broadcasted_iota(jnp.int32, (8, K), 1)
    inf = jnp.float32(np.inf)

    def inner(k, carry):
        d, acc = carry
        m = jnp.min(d, axis=1, keepdims=True)
        idx = jnp.min(jnp.where(d == m, lane, n), axis=1, keepdims=True)
        acc = acc + (kcol == k).astype(jnp.int32) * idx.astype(jnp.int32)
        return jnp.where(lane == idx, inf, d), acc

    def outer(r, _):
        d = d_scr[pl.ds(r * 8, 8), :]
        acc0 = kcol * 0
        _, acc = jax.lax.fori_loop(0, K, inner, (d, acc0))
        out_ref[0, pl.ds(r * 8, 8), :] = acc
        return 0

    jax.lax.fori_loop(0, _TSK // 8, outer, 0)


def _knn(xyz, new_xyz):
    n = xyz.shape[1]
    s = new_xyz.shape[1]
    ts = min(_TSK, s)
    xt = jnp.transpose(xyz, (0, 2, 1))
    return pl.pallas_call(
        partial(_knn_body, n=n),
        grid=(B, s // ts),
        in_specs=[
            pl.BlockSpec((1, 3, n), lambda b, i: (b, 0, 0)),
            pl.BlockSpec((1, ts, 3), lambda b, i: (b, i, 0)),
        ],
        out_specs=pl.BlockSpec((1, ts, K), lambda b, i: (b, i, 0)),
        out_shape=jax.ShapeDtypeStruct((B, s, K), jnp.int32),
        scratch_shapes=[pltpu.VMEM((ts, n), jnp.float32)],
    )(xt, new_xyz)


_TSG = 32


def _gather_body(idx_ref, aug_ref, fk_ref, xk_ref, *, c):
    def srow(s, _):
        for k in range(K):
            i = idx_ref[0, s, k]
            row = aug_ref[0, pl.ds(i, 1), :]
            fk_ref[0, s, k:k + 1, :] = row[:, :c]
            xk_ref[0, s, k:k + 1, :] = row[:, c:c + 3]
        return 0

    jax.lax.fori_loop(0, _TSG, srow, 0)


def _gather_knn(feat, xyz, idx):
    c = feat.shape[-1]
    n = feat.shape[1]
    s = idx.shape[1]
    aug = jnp.concatenate([feat, xyz], axis=-1)
    return pl.pallas_call(
        partial(_gather_body, c=c),
        grid=(B, s // _TSG),
        in_specs=[
            pl.BlockSpec((1, _TSG, K), lambda b, i: (b, i, 0),
                         memory_space=pltpu.SMEM),
            pl.BlockSpec((1, n, c + 3), lambda b, i: (b, 0, 0)),
        ],
        out_specs=[
            pl.BlockSpec((1, _TSG, K, c), lambda b, i: (b, i, 0, 0)),
            pl.BlockSpec((1, _TSG, K, 3), lambda b, i: (b, i, 0, 0)),
        ],
        out_shape=[
            jax.ShapeDtypeStruct((B, s, K, c), jnp.float32),
            jax.ShapeDtypeStruct((B, s, K, 3), jnp.float32),
        ],
    )(idx, aug)


_TSF = 8


def _fuse_body(fk_ref, xk_ref, fs_ref, xs_ref, den_ref, m0_ref, m1_ref,
               m2_ref, de_ref, issin_ref, out_ref, *, c):
    c2 = 2 * c
    fk = fk_ref[0]
    xk = xk_ref[0]
    fs = fs_ref[0]
    xs = xs_ref[0]
    xden = den_ref[0:1, 0:1]
    fden = den_ref[0:1, 1:2]
    xkn = (xk - xs[:, None, :]) / xden[None]
    fkn = (fk - fs[:, None, :]) / fden[None]
    m0 = m0_ref[:, :].reshape(1, 1, c2)
    m1 = m1_ref[:, :].reshape(1, 1, c2)
    m2 = m2_ref[:, :].reshape(1, 1, c2)
    de = de_ref[:, :].reshape(1, 1, c2)
    issin = issin_ref[:, :].reshape(1, 1, c2) != 0.0
    xsel = xkn[:, :, 0:1] * m0 + xkn[:, :, 1:2] * m1 + xkn[:, :, 2:3] * m2
    div = (BETA * xsel) / de
    pe = jnp.where(issin, jnp.sin(div), jnp.cos(div))
    pe1 = pe[:, :, :c]
    pe2 = pe[:, :, c:]
    w1 = (fkn + pe1) * pe1
    w2 = (fs[:, None, :] + pe2) * pe2
    agg1 = jnp.max(w1, axis=1) + jnp.mean(w1, axis=1)
    agg2 = jnp.max(w2, axis=1) + jnp.mean(w2, axis=1)
    out_ref[0] = jnp.concatenate([agg1, agg2], axis=-1)


def _fuse(fk, xk, fs, xs, dens, maps, c):
    s = fs.shape[1]
    c2 = 2 * c
    m0, m1, m2, de, issin = (jnp.asarray(a) for a in maps)
    return pl.pallas_call(
        partial(_fuse_body, c=c),
        grid=(B, s // _TSF),
        in_specs=[
            pl.BlockSpec((1, _TSF, K, c), lambda b, i: (b, i, 0, 0)),
            pl.BlockSpec((1, _TSF, K, 3), lambda b, i: (b, i, 0, 0)),
            pl.BlockSpec((1, _TSF, c), lambda b, i: (b, i, 0)),
            pl.BlockSpec((1, _TSF, 3), lambda b, i: (b, i, 0)),
            pl.BlockSpec((1, 2), lambda b, i: (0, 0)),
            pl.BlockSpec((1, c2), lambda b, i: (0, 0)),
            pl.BlockSpec((1, c2), lambda b, i: (0, 0)),
            pl.BlockSpec((1, c2), lambda b, i: (0, 0)),
            pl.BlockSpec((1, c2), lambda b, i: (0, 0)),
            pl.BlockSpec((1, c2), lambda b, i: (0, 0)),
        ],
        out_specs=pl.BlockSpec((1, _TSF, c2), lambda b, i: (b, i, 0)),
        out_shape=jax.ShapeDtypeStruct((B, s, c2), jnp.float32),
    )(fk, xk, fs, xs, dens, m0, m1, m2, de, issin)


def _index_points(points, idx):
    bidx = jnp.arange(B).reshape((B,) + (1,) * (idx.ndim - 1))
    return points[bidx, idx]


def _spe_embed_jnp(xyz, out_dim):
    in_dim = 3
    feat_dim = math.ceil(out_dim / (in_dim * 2))
    feat_num = feat_dim * 2 * in_dim
    out_idx = jnp.asarray(np.linspace(0, feat_num - 1, out_dim).astype(np.int32))
    feat_val = jnp.arange(feat_dim, dtype=jnp.float32)
    dim_embed = jnp.power(jnp.float32(ALPHA), feat_val / feat_dim)
    embeds = []
    for i in range(in_dim):
        tmp = BETA * xyz[..., i:i + 1]
        div = tmp / dim_embed
        e = jnp.stack([jnp.sin(div), jnp.cos(div)], axis=-1)
        e = e.reshape(e.shape[:-2] + (feat_dim * 2,))
        embeds.append(e)
    pe = jnp.concatenate(embeds, axis=-1)
    return jnp.take(pe, out_idx, axis=-1)


def _bn_gelu(x, gamma, beta_p):
    m = jnp.mean(x, axis=(0, 1), keepdims=True)
    v = jnp.var(x, axis=(0, 1), keepdims=True)
    y = gamma * (x - m) / jnp.sqrt(v + 1e-5) + beta_p
    return jax.nn.gelu(y, approximate=False)


def _forward(xyz, bn_params):
    feat = _spe_embed_jnp(xyz, INIT_DIM)
    out_dim = INIT_DIM
    stage_points = N0
    for i in range(4):
        out_dim = out_dim * 2
        stage_points = stage_points // 2
        c = out_dim // 2
        fps_idx = _fps(xyz, stage_points)
        xyz_s = _index_points(xyz, fps_idx)
        feat_s = _index_points(feat, fps_idx)
        idx_knn = _knn(xyz, xyz_s)
        feat_knn, xyz_knn = _gather_knn(feat, xyz, idx_knn)
        xyz_std = jnp.std(xyz_knn - xyz_s[:, :, None, :], ddof=1)
        feat_std = jnp.std(feat_knn - feat_s[:, :, None, :], ddof=1)
        dens = jnp.stack([xyz_std + 1e-5, feat_std + 1e-5]).reshape(1, 2)
        maps = _spe_maps(out_dim)
        agg = _fuse(feat_knn, xyz_knn, feat_s, xyz_s, dens, maps, c)
        feat = _bn_gelu(agg, bn_params[i][0], bn_params[i][1])
        xyz = xyz_s
    return jnp.max(feat, axis=-2) + jnp.mean(feat, axis=-2)


@jax.jit
def kernel(xyz, bn_w0, bn_b0, bn_w1, bn_b1, bn_w2, bn_b2, bn_w3, bn_b3):
    bn_params = [(bn_w0, bn_b0), (bn_w1, bn_b1), (bn_w2, bn_b2),
                 (bn_w3, bn_b3)]
    return _forward(xyz, bn_params)


# speedup vs baseline: 2.0257x; 1.0243x over previous
import math
from functools import partial

import jax
import jax.numpy as jnp
import numpy as np
from jax.experimental import pallas as pl
from jax.experimental.pallas import tpu as pltpu

B = 8
N0 = 2048
INIT_DIM = 64
K = 24
ALPHA = 1000.0
BETA = 100.0
STAGE_DIMS = [128, 256, 512, 1024]


def _spe_maps(out_dim):
    in_dim = 3
    feat_dim = math.ceil(out_dim / (in_dim * 2))
    feat_num = feat_dim * 2 * in_dim
    out_idx = np.linspace(0, feat_num - 1, out_dim).astype(np.int32)
    dim_embed = np.power(np.float32(ALPHA),
                         np.arange(feat_dim, dtype=np.float32) / feat_dim)
    coord = out_idx // (2 * feat_dim)
    rem = out_idx % (2 * feat_dim)
    freq = rem // 2
    issin = (rem % 2 == 0)
    m0 = (coord == 0).astype(np.float32)
    m1 = (coord == 1).astype(np.float32)
    m2 = (coord == 2).astype(np.float32)
    de = dim_embed[freq].astype(np.float32)
    return (m0.reshape(1, out_dim), m1.reshape(1, out_dim),
            m2.reshape(1, out_dim), de.reshape(1, out_dim),
            issin.astype(np.float32).reshape(1, out_dim))


def _fps_body(xt_ref, cent_ref, *, npoint, n):
    X = xt_ref[0]
    Y = xt_ref[1]
    Z = xt_ref[2]
    lane = jax.lax.broadcasted_iota(jnp.int32, (B, n), 1)
    col = jax.lax.broadcasted_iota(jnp.int32, (B, npoint), 1)

    def body(i, carry):
        dist, far, cent = carry
        cent = cent + (col == i).astype(jnp.int32) * far
        mask = lane == far
        cx = jnp.sum(jnp.where(mask, X, 0.0), axis=1, keepdims=True)
        cy = jnp.sum(jnp.where(mask, Y, 0.0), axis=1, keepdims=True)
        cz = jnp.sum(jnp.where(mask, Z, 0.0), axis=1, keepdims=True)
        dx = X - cx
        dy = Y - cy
        dz = Z - cz
        d = dx * dx + dy * dy + dz * dz
        dist = jnp.minimum(dist, d)
        m = jnp.max(dist, axis=1, keepdims=True)
        far2 = jnp.min(jnp.where(dist == m, lane, n), axis=1, keepdims=True)
        return dist, far2.astype(jnp.int32), cent

    cent_ref[:, :] = jnp.zeros((B, npoint), jnp.int32)
    cent0 = cent_ref[:, :]
    dist0 = jnp.minimum(X * X * 0.0 + 1e10, 1e10)
    far0 = jnp.zeros((B, 1), jnp.int32)
    _, _, cent = jax.lax.fori_loop(0, npoint, body, (dist0, far0, cent0))
    cent_ref[:, :] = cent


def _fps(xyz, npoint):
    n = xyz.shape[1]
    xt = jnp.transpose(xyz, (2, 0, 1))
    return pl.pallas_call(
        partial(_fps_body, npoint=npoint, n=n),
        out_shape=jax.ShapeDtypeStruct((B, npoint), jnp.int32),
    )(xt)


_TSK = 128


def _knn_body(xt_ref, q_ref, out_ref, d_scr, *, n):
    xt = xt_ref[0]
    xr = xt[0:1, :]
    yr = xt[1:2, :]
    zr = xt[2:3, :]
    q = q_ref[0]
    qx = q[:, 0:1]
    qy = q[:, 1:2]
    qz = q[:, 2:3]
    xsq = xr * xr + yr * yr + zr * zr
    qsq = qx * qx + qy * qy + qz * qz
    dot = jax.lax.dot_general(q, xt, (((1,), (0,)), ((), ())),
                              preferred_element_type=jnp.float32)
    d_scr[:, :] = (qsq - 2.0 * dot) + xsq
    lane = jax.lax.broadcasted_iota(jnp.int32, (8, n), 1)
    kcol = jax.lax.broadcasted_iota(jnp.int32, (8, K), 1)
    inf = jnp.float32(np.inf)

    def inner(k, carry):
        d, acc = carry
        m = jnp.min(d, axis=1, keepdims=True)
        idx = jnp.min(jnp.where(d == m, lane, n), axis=1, keepdims=True)
        acc = acc + (kcol == k).astype(jnp.int32) * idx.astype(jnp.int32)
        return jnp.where(lane == idx, inf, d), acc

    def outer(r, _):
        d = d_scr[pl.ds(r * 8, 8), :]
        acc0 = kcol * 0
        _, acc = jax.lax.fori_loop(0, K, inner, (d, acc0))
        out_ref[0, pl.ds(r * 8, 8), :] = acc
        return 0

    jax.lax.fori_loop(0, _TSK // 8, outer, 0)


def _knn(xyz, new_xyz):
    n = xyz.shape[1]
    s = new_xyz.shape[1]
    ts = min(_TSK, s)
    xt = jnp.transpose(xyz, (0, 2, 1))
    return pl.pallas_call(
        partial(_knn_body, n=n),
        grid=(B, s // ts),
        in_specs=[
            pl.BlockSpec((1, 3, n), lambda b, i: (b, 0, 0)),
            pl.BlockSpec((1, ts, 3), lambda b, i: (b, i, 0)),
        ],
        out_specs=pl.BlockSpec((1, ts, K), lambda b, i: (b, i, 0)),
        out_shape=jax.ShapeDtypeStruct((B, s, K), jnp.int32),
        scratch_shapes=[pltpu.VMEM((ts, n), jnp.float32)],
    )(xt, new_xyz)


_TSG = 32


def _gather_body(idx_ref, aug_ref, fk_ref, xk_ref, *, c):
    def srow(s, _):
        for k in range(K):
            i = idx_ref[0, s, k]
            row = aug_ref[0, pl.ds(i, 1), :]
            fk_ref[0, s, k:k + 1, :] = row[:, :c]
            xk_ref[0, s, k:k + 1, :] = row[:, c:c + 3]
        return 0

    jax.lax.fori_loop(0, _TSG, srow, 0)


def _gather_knn(feat, xyz, idx):
    c = feat.shape[-1]
    n = feat.shape[1]
    s = idx.shape[1]
    aug = jnp.concatenate([feat, xyz], axis=-1)
    return pl.pallas_call(
        partial(_gather_body, c=c),
        grid=(B, s // _TSG),
        in_specs=[
            pl.BlockSpec((1, _TSG, K), lambda b, i: (b, i, 0),
                         memory_space=pltpu.SMEM),
            pl.BlockSpec((1, n, c + 3), lambda b, i: (b, 0, 0)),
        ],
        out_specs=[
            pl.BlockSpec((1, _TSG, K, c), lambda b, i: (b, i, 0, 0)),
            pl.BlockSpec((1, _TSG, K, 3), lambda b, i: (b, i, 0, 0)),
        ],
        out_shape=[
            jax.ShapeDtypeStruct((B, s, K, c), jnp.float32),
            jax.ShapeDtypeStruct((B, s, K, 3), jnp.float32),
        ],
    )(idx, aug)


_TSF = 32


def _fuse_body(fk_ref, xk_ref, fs_ref, xs_ref, den_ref, m0_ref, m1_ref,
               m2_ref, de_ref, issin_ref, out_ref, *, c):
    c2 = 2 * c
    fk = fk_ref[0]
    xk = xk_ref[0]
    fs = fs_ref[0]
    xs = xs_ref[0]
    xden = den_ref[0:1, 0:1]
    fden = den_ref[0:1, 1:2]
    xkn = (xk - xs[:, None, :]) / xden[None]
    fkn = (fk - fs[:, None, :]) / fden[None]
    m0 = m0_ref[:, :].reshape(1, 1, c2)
    m1 = m1_ref[:, :].reshape(1, 1, c2)
    m2 = m2_ref[:, :].reshape(1, 1, c2)
    de = de_ref[:, :].reshape(1, 1, c2)
    issin = issin_ref[:, :].reshape(1, 1, c2) != 0.0
    xsel = xkn[:, :, 0:1] * m0 + xkn[:, :, 1:2] * m1 + xkn[:, :, 2:3] * m2
    div = (BETA * xsel) / de
    pe = jnp.where(issin, jnp.sin(div), jnp.cos(div))
    pe1 = pe[:, :, :c]
    pe2 = pe[:, :, c:]
    w1 = (fkn + pe1) * pe1
    w2 = (fs[:, None, :] + pe2) * pe2
    agg1 = jnp.max(w1, axis=1) + jnp.mean(w1, axis=1)
    agg2 = jnp.max(w2, axis=1) + jnp.mean(w2, axis=1)
    out_ref[0] = jnp.concatenate([agg1, agg2], axis=-1)


def _fuse(fk, xk, fs, xs, dens, maps, c):
    s = fs.shape[1]
    c2 = 2 * c
    m0, m1, m2, de, issin = (jnp.asarray(a) for a in maps)
    return pl.pallas_call(
        partial(_fuse_body, c=c),
        grid=(B, s // _TSF),
        in_specs=[
            pl.BlockSpec((1, _TSF, K, c), lambda b, i: (b, i, 0, 0)),
            pl.BlockSpec((1, _TSF, K, 3), lambda b, i: (b, i, 0, 0)),
            pl.BlockSpec((1, _TSF, c), lambda b, i: (b, i, 0)),
            pl.BlockSpec((1, _TSF, 3), lambda b, i: (b, i, 0)),
            pl.BlockSpec((1, 2), lambda b, i: (0, 0)),
            pl.BlockSpec((1, c2), lambda b, i: (0, 0)),
            pl.BlockSpec((1, c2), lambda b, i: (0, 0)),
            pl.BlockSpec((1, c2), lambda b, i: (0, 0)),
            pl.BlockSpec((1, c2), lambda b, i: (0, 0)),
            pl.BlockSpec((1, c2), lambda b, i: (0, 0)),
        ],
        out_specs=pl.BlockSpec((1, _TSF, c2), lambda b, i: (b, i, 0)),
        out_shape=jax.ShapeDtypeStruct((B, s, c2), jnp.float32),
    )(fk, xk, fs, xs, dens, m0, m1, m2, de, issin)


def _index_points(points, idx):
    bidx = jnp.arange(B).reshape((B,) + (1,) * (idx.ndim - 1))
    return points[bidx, idx]


def _spe_embed_jnp(xyz, out_dim):
    in_dim = 3
    feat_dim = math.ceil(out_dim / (in_dim * 2))
    feat_num = feat_dim * 2 * in_dim
    out_idx = jnp.asarray(np.linspace(0, feat_num - 1, out_dim).astype(np.int32))
    feat_val = jnp.arange(feat_dim, dtype=jnp.float32)
    dim_embed = jnp.power(jnp.float32(ALPHA), feat_val / feat_dim)
    embeds = []
    for i in range(in_dim):
        tmp = BETA * xyz[..., i:i + 1]
        div = tmp / dim_embed
        e = jnp.stack([jnp.sin(div), jnp.cos(div)], axis=-1)
        e = e.reshape(e.shape[:-2] + (feat_dim * 2,))
        embeds.append(e)
    pe = jnp.concatenate(embeds, axis=-1)
    return jnp.take(pe, out_idx, axis=-1)


def _bn_gelu(x, gamma, beta_p):
    m = jnp.mean(x, axis=(0, 1), keepdims=True)
    v = jnp.var(x, axis=(0, 1), keepdims=True)
    y = gamma * (x - m) / jnp.sqrt(v + 1e-5) + beta_p
    return jax.nn.gelu(y, approximate=False)


def _forward(xyz, bn_params):
    feat = _spe_embed_jnp(xyz, INIT_DIM)
    out_dim = INIT_DIM
    stage_points = N0
    for i in range(4):
        out_dim = out_dim * 2
        stage_points = stage_points // 2
        c = out_dim // 2
        fps_idx = _fps(xyz, stage_points)
        xyz_s = _index_points(xyz, fps_idx)
        feat_s = _index_points(feat, fps_idx)
        idx_knn = _knn(xyz, xyz_s)
        feat_knn, xyz_knn = _gather_knn(feat, xyz, idx_knn)
        xyz_std = jnp.std(xyz_knn - xyz_s[:, :, None, :], ddof=1)
        feat_std = jnp.std(feat_knn - feat_s[:, :, None, :], ddof=1)
        dens = jnp.stack([xyz_std + 1e-5, feat_std + 1e-5]).reshape(1, 2)
        maps = _spe_maps(out_dim)
        agg = _fuse(feat_knn, xyz_knn, feat_s, xyz_s, dens, maps, c)
        feat = _bn_gelu(agg, bn_params[i][0], bn_params[i][1])
        xyz = xyz_s
    return jnp.max(feat, axis=-2) + jnp.mean(feat, axis=-2)


@jax.jit
def kernel(xyz, bn_w0, bn_b0, bn_w1, bn_b1, bn_w2, bn_b2, bn_w3, bn_b3):
    bn_params = [(bn_w0, bn_b0), (bn_w1, bn_b1), (bn_w2, bn_b2),
                 (bn_w3, bn_b3)]
    return _forward(xyz, bn_params)
